# Initial kernel scaffold; baseline (speedup 1.0000x reference)
import jax
import jax.numpy as jnp
from jax.experimental import pallas as pl


def kernel(atomic_numbers, distances, idx_i, idx_j, c6ab, rcov, r2r4, d3_s6, d3_s8, d3_a1, d3_a2):
    raise NotImplementedError("write your pallas kernel here")


# speedup vs baseline: 185.3471x; 185.3471x over previous
import functools

import jax
import jax.numpy as jnp
from jax import lax
from jax.experimental import pallas as pl
from jax.experimental.pallas import tpu as pltpu
from jax.experimental.pallas import tpu_sc as plsc

N_ATOMS = 50000
N_PAIRS = 1600000
N_ELEM = 95
CUTOFF = 10.0
WIDTH = 2.0
CUTON = CUTOFF - WIDTH
K1 = 16.0
K3 = -4.0

NC = 2
NS = 16
NT = NC * NS
NAP = 50176
ASL = NAP // NS
PT = 50176
NPP = NT * PT
PROWS = NPP // 128

R1 = 8
C1 = R1 * 128
R2 = 8
C2 = R2 * 128
SUB2 = 256
TROW = 128


def _smoother(d):
    x = (CUTOFF - d) * (1.0 / WIDTH)
    poly = ((6.0 * x - 15.0) * x + 10.0) * x * x * x
    return jnp.where(d < CUTON, 1.0, jnp.where(d >= CUTOFF, 0.0, poly))


def _phase1_body(atoms_h, ii_h, ii2_h, jj_h, d_h, rcov_h, u_h,
                 ncpart, cls_o, srr_o,
                 atoms_v, rcov_v, u_v, ii_v, ii2_v, jj_v, d_v,
                 vals_v, cls_v, srr_v, zbuf, acc):
    cid = lax.axis_index("c")
    sid = lax.axis_index("s")
    wid = cid * NS + sid

    pltpu.sync_copy(atoms_h, atoms_v)
    pltpu.sync_copy(rcov_h, rcov_v)
    pltpu.sync_copy(u_h, u_v)
    @pl.loop(0, ASL // 16)
    def _z(k):
        zbuf[pl.ds(k * 16, 16)] = jnp.zeros((16,), jnp.float32)

    pltpu.sync_copy(zbuf, acc.at[pl.ds(sid * ASL, ASL)])
    plsc.subcore_barrier()

    pair_base = wid * PT
    row_base = wid * (PT // 128)
    iota = lax.iota(jnp.int32, 16)

    @pl.loop(0, PT // C1)
    def _chunk(c):
        off = pair_base + c * C1
        pltpu.sync_copy(ii_h.at[pl.ds(off, C1)], ii_v)
        pltpu.sync_copy(ii2_h.at[pl.ds(row_base + c * R1, R1)], ii2_v)
        pltpu.sync_copy(jj_h.at[pl.ds(off, C1)], jj_v)
        pltpu.sync_copy(d_h.at[pl.ds(off, C1)], d_v)

        @pl.loop(0, C1 // 16)
        def _grp(k):
            s = k * 16
            ii = ii_v[pl.ds(s, 16)]
            jj = jj_v[pl.ds(s, 16)]
            d = d_v[pl.ds(s, 16)]
            zi = plsc.load_gather(atoms_v, [ii])
            zj = plsc.load_gather(atoms_v, [jj])
            rco = (plsc.load_gather(rcov_v, [zi]) +
                   plsc.load_gather(rcov_v, [zj]))
            damp = 1.0 / (1.0 + jnp.exp(-K1 * (rco / d - 1.0)))
            vals_v[pl.ds(s, 16)] = damp * _smoother(d)
            cls_v[pl.ds(s, 16)] = zi * N_ELEM + zj
            srr_v[pl.ds(s, 16)] = (plsc.load_gather(u_v, [zi]) *
                                   plsc.load_gather(u_v, [zj]))

        for g in range(R1):
            pltpu.sync_copy(vals_v.at[pl.ds(g * 128, 128)],
                            acc.at[ii2_v.at[g]], add=True)
        pltpu.sync_copy(cls_v, cls_o.at[pl.ds(off, C1)])
        pltpu.sync_copy(srr_v, srr_o.at[pl.ds(off, C1)])

    plsc.subcore_barrier()
    pltpu.sync_copy(acc.at[pl.ds(sid * ASL, ASL)], zbuf)
    pltpu.sync_copy(zbuf, ncpart.at[pl.ds(cid * NAP + sid * ASL, ASL)])


def _phase2_body(table_h, nc_h, ii_h, ii2_h, jj_h, d_h, srr_h, cls2_h,
                 params_h,
                 epart,
                 nc_v, params_v, ii_v, ii2_v, jj_v, d_v, srr_v, ev_v,
                 cls2_v, rows_v, zbuf, sem, acc):
    cid = lax.axis_index("c")
    sid = lax.axis_index("s")
    wid = cid * NS + sid

    pltpu.sync_copy(nc_h, nc_v)
    pltpu.sync_copy(params_h, params_v)
    @pl.loop(0, ASL // 16)
    def _z(k):
        zbuf[pl.ds(k * 16, 16)] = jnp.zeros((16,), jnp.float32)

    pltpu.sync_copy(zbuf, acc.at[pl.ds(sid * ASL, ASL)])
    plsc.subcore_barrier()

    pv = params_v[pl.ds(0, 16)]
    s6 = pv[0]
    s8 = pv[1]
    a1 = pv[2]
    a2 = pv[3]
    pair_base = wid * PT
    row_base = wid * (PT // 128)
    iota = lax.iota(jnp.int32, 16)

    @pl.loop(0, PT // C2)
    def _chunk(c):
        off = pair_base + c * C2
        pltpu.sync_copy(cls2_h.at[pl.ds(row_base + c * R2, R2)], cls2_v)
        pltpu.sync_copy(ii_h.at[pl.ds(off, C2)], ii_v)
        pltpu.sync_copy(ii2_h.at[pl.ds(row_base + c * R2, R2)], ii2_v)
        pltpu.sync_copy(jj_h.at[pl.ds(off, C2)], jj_v)
        pltpu.sync_copy(d_h.at[pl.ds(off, C2)], d_v)
        pltpu.sync_copy(srr_h.at[pl.ds(off, C2)], srr_v)
        for u in range(C2 // SUB2):
            for h in range(SUB2 // 128):
                g = u * (SUB2 // 128) + h
                pltpu.async_copy(table_h.at[cls2_v.at[g]],
                                 rows_v.at[pl.ds(h * 128, 128)], sem).wait()

            @pl.loop(0, SUB2 // 16)
            def _grp(k):
                s = u * SUB2 + k * 16
                ii = ii_v[pl.ds(s, 16)]
                jj = jj_v[pl.ds(s, 16)]
                d = d_v[pl.ds(s, 16)]
                srr = srr_v[pl.ds(s, 16)]
                nci = plsc.load_gather(nc_v, [ii])
                ncj = plsc.load_gather(nc_v, [jj])
                pr = k * 16 + iota
                rsum = jnp.zeros((16,), jnp.float32)
                csum = jnp.zeros((16,), jnp.float32)
                for t in range(25):
                    c0 = plsc.load_gather(rows_v, [pr, jnp.full((16,), t, jnp.int32)])
                    c1 = plsc.load_gather(rows_v, [pr, jnp.full((16,), 25 + t, jnp.int32)])
                    c2 = plsc.load_gather(rows_v, [pr, jnp.full((16,), 50 + t, jnp.int32)])
                    dr1 = c1 - nci
                    dr2 = c2 - ncj
                    w = jnp.exp(K3 * (dr1 * dr1 + dr2 * dr2))
                    m = c0 > 0.0
                    rsum = rsum + jnp.where(m, w, 0.0)
                    csum = csum + jnp.where(m, w * c0, 0.0)
                c6 = jnp.where(rsum > 0.0, csum / rsum, -1e38)
                c8 = c6 * srr * srr
                tmp = a1 * srr + a2
                tmp2 = tmp * tmp
                tmp6 = tmp2 * tmp2 * tmp2
                tmp8 = tmp6 * tmp2
                d2 = d * d
                d6 = d2 * d2 * d2
                d8 = d6 * d2
                sw = _smoother(d)
                ev_v[pl.ds(s, 16)] = -0.5 * sw * (s6 * c6 / (d6 + tmp6) +
                                                  s8 * c8 / (d8 + tmp8))

        for g in range(R2):
            pltpu.sync_copy(ev_v.at[pl.ds(g * 128, 128)],
                            acc.at[ii2_v.at[g]], add=True)

    plsc.subcore_barrier()
    pltpu.sync_copy(acc.at[pl.ds(sid * ASL, ASL)], zbuf)
    pltpu.sync_copy(zbuf, epart.at[pl.ds(cid * NAP + sid * ASL, ASL)])


def _combine_body(a_ref, o_ref):
    o_ref[...] = a_ref[0] + a_ref[1]


def _combine(parts):
    out = pl.pallas_call(
        _combine_body,
        out_shape=jax.ShapeDtypeStruct((NAP // 128, 128), jnp.float32),
    )(parts.reshape(2, NAP // 128, 128))
    return out.reshape(NAP)


@jax.jit
def kernel(atomic_numbers, distances, idx_i, idx_j, c6ab, rcov, r2r4,
           d3_s6, d3_s8, d3_a1, d3_a2):
    f32 = jnp.float32
    i32 = jnp.int32
    pad_p = NPP - N_PAIRS
    ii = jnp.pad(idx_i.astype(i32), (0, pad_p))
    jj = jnp.pad(idx_j.astype(i32), (0, pad_p))
    d = jnp.pad(distances.astype(f32), (0, pad_p), constant_values=11.0)
    ii2 = ii.reshape(PROWS, 128)
    atoms = jnp.pad(atomic_numbers.astype(i32), (0, NAP - N_ATOMS))
    rcov96 = jnp.pad(rcov.astype(f32), (0, 96 - N_ELEM))
    u96 = jnp.pad((3.0 ** 0.25) * jnp.sqrt(r2r4.astype(f32)),
                  (0, 96 - N_ELEM))
    table = c6ab.astype(f32).transpose(0, 1, 4, 2, 3).reshape(N_ELEM * N_ELEM, 75)
    table = jnp.pad(table, ((0, 0), (0, TROW - 75)))
    params = jnp.pad(jnp.stack([d3_s6, d3_s8, d3_a1, d3_a2]).astype(f32),
                     (0, 12))

    mesh = plsc.VectorSubcoreMesh(core_axis_name="c", subcore_axis_name="s",
                                  num_cores=NC, num_subcores=NS)
    cparams = pltpu.CompilerParams(needs_layout_passes=False)

    p1 = pl.kernel(
        _phase1_body,
        out_type=[
            jax.ShapeDtypeStruct((NC * NAP,), f32),
            jax.ShapeDtypeStruct((NPP,), i32),
            jax.ShapeDtypeStruct((NPP,), f32),
        ],
        mesh=mesh,
        compiler_params=cparams,
        scratch_types=[
            pltpu.VMEM((NAP,), i32),
            pltpu.VMEM((96,), f32),
            pltpu.VMEM((96,), f32),
            pltpu.VMEM((C1,), i32),
            pltpu.VMEM((R1, 128), i32),
            pltpu.VMEM((C1,), i32),
            pltpu.VMEM((C1,), f32),
            pltpu.VMEM((C1,), f32),
            pltpu.VMEM((C1,), i32),
            pltpu.VMEM((C1,), f32),
            pltpu.VMEM((ASL,), f32),
            pltpu.VMEM_SHARED((NAP,), f32),
        ],
    )
    ncpart, cls, srr = p1(atoms, ii, ii2, jj, d, rcov96, u96)
    nc = _combine(ncpart)

    p2 = pl.kernel(
        _phase2_body,
        out_type=jax.ShapeDtypeStruct((NC * NAP,), f32),
        mesh=mesh,
        compiler_params=cparams,
        scratch_types=[
            pltpu.VMEM((NAP,), f32),
            pltpu.VMEM((16,), f32),
            pltpu.VMEM((C2,), i32),
            pltpu.VMEM((R2, 128), i32),
            pltpu.VMEM((C2,), i32),
            pltpu.VMEM((C2,), f32),
            pltpu.VMEM((C2,), f32),
            pltpu.VMEM((C2,), f32),
            pltpu.VMEM((R2, 128), i32),
            pltpu.VMEM((SUB2, TROW), f32),
            pltpu.VMEM((ASL,), f32),
            pltpu.SemaphoreType.DMA,
            pltpu.VMEM_SHARED((NAP,), f32),
        ],
    )
    epart = p2(table, nc, ii, ii2, jj, d, srr, cls.reshape(PROWS, 128),
               params)
    return _combine(epart)[:N_ATOMS]


# speedup vs baseline: 290.0145x; 1.5647x over previous
import jax
import jax.numpy as jnp
from jax import lax
from jax.experimental import pallas as pl
from jax.experimental.pallas import tpu as pltpu
from jax.experimental.pallas import tpu_sc as plsc

N_ATOMS = 50000
N_PAIRS = 1600000
N_ELEM = 95
CUTOFF = 10.0
WIDTH = 2.0
CUTON = CUTOFF - WIDTH
K1 = 16.0
K3 = -4.0

NC = 2
NS = 16
NT = NC * NS
NAP = 50176
ASL = NAP // NS
PT = 50176
NPP = NT * PT
PROWS = NPP // 128

R1 = 8
C1 = R1 * 128
R2 = 8
C2 = R2 * 128
SUB2 = 256
TROW = 80


def _smoother(d):
    x = (CUTOFF - d) * (1.0 / WIDTH)
    poly = ((6.0 * x - 15.0) * x + 10.0) * x * x * x
    return jnp.where(d < CUTON, 1.0, jnp.where(d >= CUTOFF, 0.0, poly))


def _phase1_body(atoms_h, ii_h, jj_h, d_h, rcov_h, u_h,
                 ncpart, cls_o, srr_o,
                 atoms_v, rcov_v, u_v, ii_v, jj_v, d_v,
                 vals_v, cls_v, srr_v, zbuf, acc):
    cid = lax.axis_index("c")
    sid = lax.axis_index("s")
    wid = cid * NS + sid

    pltpu.sync_copy(atoms_h, atoms_v)
    pltpu.sync_copy(rcov_h, rcov_v)
    pltpu.sync_copy(u_h, u_v)

    @pl.loop(0, ASL // 16)
    def _z(k):
        zbuf[pl.ds(k * 16, 16)] = jnp.zeros((16,), jnp.float32)

    pltpu.sync_copy(zbuf, acc.at[pl.ds(sid * ASL, ASL)])
    plsc.subcore_barrier()

    row_base = wid * (PT // 128)

    @pl.loop(0, PT // C1)
    def _chunk(c):
        roff = row_base + c * R1
        pltpu.sync_copy(ii_h.at[pl.ds(roff, R1)], ii_v)
        pltpu.sync_copy(jj_h.at[pl.ds(roff, R1)], jj_v)
        pltpu.sync_copy(d_h.at[pl.ds(roff, R1)], d_v)

        for g in range(R1):
            @pl.loop(0, 8)
            def _grp(k):
                s = k * 16
                ii = ii_v[g, pl.ds(s, 16)]
                jj = jj_v[g, pl.ds(s, 16)]
                d = d_v[g, pl.ds(s, 16)]
                zi = plsc.load_gather(atoms_v, [ii])
                zj = plsc.load_gather(atoms_v, [jj])
                rco = (plsc.load_gather(rcov_v, [zi]) +
                       plsc.load_gather(rcov_v, [zj]))
                damp = 1.0 / (1.0 + jnp.exp(-K1 * (rco / d - 1.0)))
                vals_v[g, pl.ds(s, 16)] = damp * _smoother(d)
                cls_v[g, pl.ds(s, 16)] = zi * N_ELEM + zj
                srr_v[g, pl.ds(s, 16)] = (plsc.load_gather(u_v, [zi]) *
                                          plsc.load_gather(u_v, [zj]))

        for g in range(R1):
            pltpu.sync_copy(vals_v.at[g], acc.at[ii_v.at[g]], add=True)
        pltpu.sync_copy(cls_v, cls_o.at[pl.ds(roff, R1)])
        pltpu.sync_copy(srr_v, srr_o.at[pl.ds(roff, R1)])

    plsc.subcore_barrier()
    pltpu.sync_copy(acc.at[pl.ds(sid * ASL, ASL)], zbuf)
    pltpu.sync_copy(zbuf, ncpart.at[pl.ds(cid * NAP + sid * ASL, ASL)])


def _phase2_body(table_h, nc_h, ii_h, jj_h, d_h, srr_h, cls_h, params_h,
                 epart,
                 nc_v, params_v, ii_v, jj_v, d_v, srr_v, ev_v,
                 cls_v, rows_v, zbuf, sem, acc):
    cid = lax.axis_index("c")
    sid = lax.axis_index("s")
    wid = cid * NS + sid

    pltpu.sync_copy(nc_h, nc_v)
    pltpu.sync_copy(params_h, params_v)

    @pl.loop(0, ASL // 16)
    def _z(k):
        zbuf[pl.ds(k * 16, 16)] = jnp.zeros((16,), jnp.float32)

    pltpu.sync_copy(zbuf, acc.at[pl.ds(sid * ASL, ASL)])
    plsc.subcore_barrier()

    pv = params_v[pl.ds(0, 16)]
    s6 = pv[0]
    s8 = pv[1]
    a1 = pv[2]
    a2 = pv[3]
    row_base = wid * (PT // 128)
    iota = lax.iota(jnp.int32, 16)

    @pl.loop(0, PT // C2)
    def _chunk(c):
        roff = row_base + c * R2
        pltpu.sync_copy(cls_h.at[pl.ds(roff, R2)], cls_v)
        pltpu.sync_copy(ii_h.at[pl.ds(roff, R2)], ii_v)
        pltpu.sync_copy(jj_h.at[pl.ds(roff, R2)], jj_v)
        pltpu.sync_copy(d_h.at[pl.ds(roff, R2)], d_v)
        pltpu.sync_copy(srr_h.at[pl.ds(roff, R2)], srr_v)
        for u in range(C2 // SUB2):
            for h in range(SUB2 // 128):
                pltpu.async_copy(table_h.at[cls_v.at[u * 2 + h]],
                                 rows_v.at[pl.ds(h * 128, 128)], sem).wait()
            for h in range(SUB2 // 128):
                g = u * 2 + h

                @pl.loop(0, 8)
                def _grp(k):
                    s = k * 16
                    ii = ii_v[g, pl.ds(s, 16)]
                    jj = jj_v[g, pl.ds(s, 16)]
                    d = d_v[g, pl.ds(s, 16)]
                    srr = srr_v[g, pl.ds(s, 16)]
                    nci = plsc.load_gather(nc_v, [ii])
                    ncj = plsc.load_gather(nc_v, [jj])
                    pr = h * 128 + s + iota
                    rsum = jnp.zeros((16,), jnp.float32)
                    csum = jnp.zeros((16,), jnp.float32)
                    for t in range(25):
                        c0 = plsc.load_gather(rows_v, [pr, jnp.full((16,), t, jnp.int32)])
                        c1 = plsc.load_gather(rows_v, [pr, jnp.full((16,), 25 + t, jnp.int32)])
                        c2 = plsc.load_gather(rows_v, [pr, jnp.full((16,), 50 + t, jnp.int32)])
                        dr1 = c1 - nci
                        dr2 = c2 - ncj
                        w = jnp.exp(K3 * (dr1 * dr1 + dr2 * dr2))
                        m = c0 > 0.0
                        rsum = rsum + jnp.where(m, w, 0.0)
                        csum = csum + jnp.where(m, w * c0, 0.0)
                    c6 = jnp.where(rsum > 0.0, csum / rsum, -1e38)
                    c8 = c6 * srr * srr
                    tmp = a1 * srr + a2
                    tmp2 = tmp * tmp
                    tmp6 = tmp2 * tmp2 * tmp2
                    tmp8 = tmp6 * tmp2
                    d2 = d * d
                    d6 = d2 * d2 * d2
                    d8 = d6 * d2
                    sw = _smoother(d)
                    ev_v[g, pl.ds(s, 16)] = -0.5 * sw * (s6 * c6 / (d6 + tmp6) +
                                                         s8 * c8 / (d8 + tmp8))

        for g in range(R2):
            pltpu.sync_copy(ev_v.at[g], acc.at[ii_v.at[g]], add=True)

    plsc.subcore_barrier()
    pltpu.sync_copy(acc.at[pl.ds(sid * ASL, ASL)], zbuf)
    pltpu.sync_copy(zbuf, epart.at[pl.ds(cid * NAP + sid * ASL, ASL)])


def _combine_body(a_ref, o_ref):
    o_ref[...] = a_ref[0] + a_ref[1]


def _combine(parts):
    out = pl.pallas_call(
        _combine_body,
        out_shape=jax.ShapeDtypeStruct((NAP // 128, 128), jnp.float32),
    )(parts.reshape(2, NAP // 128, 128))
    return out.reshape(NAP)


@jax.jit
def kernel(atomic_numbers, distances, idx_i, idx_j, c6ab, rcov, r2r4,
           d3_s6, d3_s8, d3_a1, d3_a2):
    f32 = jnp.float32
    i32 = jnp.int32
    pad_p = NPP - N_PAIRS
    ii = jnp.pad(idx_i.astype(i32), (0, pad_p)).reshape(PROWS, 128)
    jj = jnp.pad(idx_j.astype(i32), (0, pad_p)).reshape(PROWS, 128)
    d = jnp.pad(distances.astype(f32), (0, pad_p),
                constant_values=11.0).reshape(PROWS, 128)
    atoms = jnp.pad(atomic_numbers.astype(i32), (0, NAP - N_ATOMS))
    rcov96 = jnp.pad(rcov.astype(f32), (0, 96 - N_ELEM))
    u96 = jnp.pad((3.0 ** 0.25) * jnp.sqrt(r2r4.astype(f32)),
                  (0, 96 - N_ELEM))
    table = c6ab.astype(f32).transpose(0, 1, 4, 2, 3).reshape(N_ELEM * N_ELEM, 75)
    table = jnp.pad(table, ((0, 0), (0, TROW - 75)))
    params = jnp.pad(jnp.stack([d3_s6, d3_s8, d3_a1, d3_a2]).astype(f32),
                     (0, 12))

    mesh = plsc.VectorSubcoreMesh(core_axis_name="c", subcore_axis_name="s",
                                  num_cores=NC, num_subcores=NS)
    cparams = pltpu.CompilerParams(needs_layout_passes=False,
                                   use_tc_tiling_on_sc=False)

    p1 = pl.kernel(
        _phase1_body,
        out_type=[
            jax.ShapeDtypeStruct((NC * NAP,), f32),
            jax.ShapeDtypeStruct((PROWS, 128), i32),
            jax.ShapeDtypeStruct((PROWS, 128), f32),
        ],
        mesh=mesh,
        compiler_params=cparams,
        scratch_types=[
            pltpu.VMEM((NAP,), i32),
            pltpu.VMEM((96,), f32),
            pltpu.VMEM((96,), f32),
            pltpu.VMEM((R1, 128), i32),
            pltpu.VMEM((R1, 128), i32),
            pltpu.VMEM((R1, 128), f32),
            pltpu.VMEM((R1, 128), f32),
            pltpu.VMEM((R1, 128), i32),
            pltpu.VMEM((R1, 128), f32),
            pltpu.VMEM((ASL,), f32),
            pltpu.VMEM_SHARED((NAP,), f32),
        ],
    )
    ncpart, cls, srr = p1(atoms, ii, jj, d, rcov96, u96)
    nc = _combine(ncpart)

    p2 = pl.kernel(
        _phase2_body,
        out_type=jax.ShapeDtypeStruct((NC * NAP,), f32),
        mesh=mesh,
        compiler_params=cparams,
        scratch_types=[
            pltpu.VMEM((NAP,), f32),
            pltpu.VMEM((16,), f32),
            pltpu.VMEM((R2, 128), i32),
            pltpu.VMEM((R2, 128), i32),
            pltpu.VMEM((R2, 128), f32),
            pltpu.VMEM((R2, 128), f32),
            pltpu.VMEM((R2, 128), f32),
            pltpu.VMEM((R2, 128), i32),
            pltpu.VMEM((SUB2, TROW), f32),
            pltpu.VMEM((ASL,), f32),
            pltpu.SemaphoreType.DMA,
            pltpu.VMEM_SHARED((NAP,), f32),
        ],
    )
    epart = p2(table, nc, ii, jj, d, srr, cls, params)
    return _combine(epart)[:N_ATOMS]


# speedup vs baseline: 358.6737x; 1.2367x over previous
import jax
import jax.numpy as jnp
from jax import lax
from jax.experimental import pallas as pl
from jax.experimental.pallas import tpu as pltpu
from jax.experimental.pallas import tpu_sc as plsc

N_ATOMS = 50000
N_PAIRS = 1600000
N_ELEM = 95
CUTOFF = 10.0
WIDTH = 2.0
CUTON = CUTOFF - WIDTH
K1 = 16.0
K3 = -4.0

NC = 2
NS = 16
NT = NC * NS
NAP = 50176
ASL = NAP // NS
PT = 50176
NPP = NT * PT
PROWS = NPP // 128

R1 = 8
C1 = R1 * 128
R2 = 8
C2 = R2 * 128
SUB2 = 256
TROW = 80


def _smoother(d):
    x = (CUTOFF - d) * (1.0 / WIDTH)
    poly = ((6.0 * x - 15.0) * x + 10.0) * x * x * x
    return jnp.where(d < CUTON, 1.0, jnp.where(d >= CUTOFF, 0.0, poly))


def _phase1_body(atoms_h, ii_h, jj_h, d_h, rcov_h, u_h,
                 ncpart, cls_o, srr_o,
                 atoms_v, rcov_v, u_v, ii_v, jj_v, d_v,
                 vals_v, cls_v, srr_v, zbuf, acc):
    cid = lax.axis_index("c")
    sid = lax.axis_index("s")
    wid = cid * NS + sid

    pltpu.sync_copy(atoms_h, atoms_v)
    pltpu.sync_copy(rcov_h, rcov_v)
    pltpu.sync_copy(u_h, u_v)

    @pl.loop(0, ASL // 16)
    def _z(k):
        zbuf[pl.ds(k * 16, 16)] = jnp.zeros((16,), jnp.float32)

    pltpu.sync_copy(zbuf, acc.at[pl.ds(sid * ASL, ASL)])
    plsc.subcore_barrier()

    row_base = wid * (PT // 128)

    @pl.loop(0, PT // C1)
    def _chunk(c):
        roff = row_base + c * R1
        pltpu.sync_copy(ii_h.at[pl.ds(roff, R1)], ii_v)
        pltpu.sync_copy(jj_h.at[pl.ds(roff, R1)], jj_v)
        pltpu.sync_copy(d_h.at[pl.ds(roff, R1)], d_v)

        for g in range(R1):
            @pl.loop(0, 8)
            def _grp(k):
                s = k * 16
                ii = ii_v[g, pl.ds(s, 16)]
                jj = jj_v[g, pl.ds(s, 16)]
                d = d_v[g, pl.ds(s, 16)]
                zi = plsc.load_gather(atoms_v, [ii])
                zj = plsc.load_gather(atoms_v, [jj])
                rco = (plsc.load_gather(rcov_v, [zi]) +
                       plsc.load_gather(rcov_v, [zj]))
                damp = 1.0 / (1.0 + jnp.exp(-K1 * (rco / d - 1.0)))
                vals_v[g, pl.ds(s, 16)] = damp * _smoother(d)
                cls_v[g, pl.ds(s, 16)] = zi * N_ELEM + zj
                srr_v[g, pl.ds(s, 16)] = (plsc.load_gather(u_v, [zi]) *
                                          plsc.load_gather(u_v, [zj]))

        for g in range(R1):
            pltpu.sync_copy(vals_v.at[g], acc.at[ii_v.at[g]], add=True)
        pltpu.sync_copy(cls_v, cls_o.at[pl.ds(roff, R1)])
        pltpu.sync_copy(srr_v, srr_o.at[pl.ds(roff, R1)])

    plsc.subcore_barrier()
    pltpu.sync_copy(acc.at[pl.ds(sid * ASL, ASL)], zbuf)
    pltpu.sync_copy(zbuf, ncpart.at[pl.ds(cid * NAP + sid * ASL, ASL)])


def _phase2_body(table_h, nc_h, ii_h, jj_h, d_h, srr_h, cls_h, params_h,
                 epart,
                 nc_v, params_v, ii_v, jj_v, d_v, srr_v, ev_v,
                 cls_v, rows_v, zbuf, sem0, sem1, sem2, sem3, acc):
    cid = lax.axis_index("c")
    sid = lax.axis_index("s")
    wid = cid * NS + sid

    pltpu.sync_copy(nc_h, nc_v)
    pltpu.sync_copy(params_h, params_v)

    @pl.loop(0, ASL // 16)
    def _z(k):
        zbuf[pl.ds(k * 16, 16)] = jnp.zeros((16,), jnp.float32)

    pltpu.sync_copy(zbuf, acc.at[pl.ds(sid * ASL, ASL)])
    plsc.subcore_barrier()

    pv = params_v[pl.ds(0, 16)]
    s6 = pv[0]
    s8 = pv[1]
    a1 = pv[2]
    a2 = pv[3]
    row_base = wid * (PT // 128)
    iota = lax.iota(jnp.int32, 16)

    @pl.loop(0, PT // C2)
    def _chunk(c):
        roff = row_base + c * R2
        pltpu.sync_copy(cls_h.at[pl.ds(roff, R2)], cls_v)
        pltpu.sync_copy(ii_h.at[pl.ds(roff, R2)], ii_v)
        pltpu.sync_copy(jj_h.at[pl.ds(roff, R2)], jj_v)
        pltpu.sync_copy(d_h.at[pl.ds(roff, R2)], d_v)
        pltpu.sync_copy(srr_h.at[pl.ds(roff, R2)], srr_v)
        sems = [sem0, sem1, sem2, sem3]
        dsc = [None] * R2
        for g in range(4):
            dsc[g] = pltpu.async_copy(table_h.at[cls_v.at[g]],
                                      rows_v.at[pl.ds((g % 4) * 128, 128)],
                                      sems[g % 4])
        for g in range(R2):
            dsc[g].wait()

            @pl.loop(0, 8)
            def _grp(k):
                s = k * 16
                ii = ii_v[g, pl.ds(s, 16)]
                jj = jj_v[g, pl.ds(s, 16)]
                d = d_v[g, pl.ds(s, 16)]
                srr = srr_v[g, pl.ds(s, 16)]
                nci = plsc.load_gather(nc_v, [ii])
                ncj = plsc.load_gather(nc_v, [jj])
                pr = (g % 4) * 128 + s + iota
                rsum = jnp.zeros((16,), jnp.float32)
                csum = jnp.zeros((16,), jnp.float32)
                for t in range(25):
                    c0 = plsc.load_gather(rows_v, [pr, jnp.full((16,), t, jnp.int32)])
                    c1 = plsc.load_gather(rows_v, [pr, jnp.full((16,), 25 + t, jnp.int32)])
                    c2 = plsc.load_gather(rows_v, [pr, jnp.full((16,), 50 + t, jnp.int32)])
                    dr1 = c1 - nci
                    dr2 = c2 - ncj
                    w = jnp.exp(K3 * (dr1 * dr1 + dr2 * dr2))
                    m = c0 > 0.0
                    rsum = rsum + jnp.where(m, w, 0.0)
                    csum = csum + jnp.where(m, w * c0, 0.0)
                c6 = jnp.where(rsum > 0.0, csum / rsum, -1e38)
                c8 = c6 * srr * srr
                tmp = a1 * srr + a2
                tmp2 = tmp * tmp
                tmp6 = tmp2 * tmp2 * tmp2
                tmp8 = tmp6 * tmp2
                d2 = d * d
                d6 = d2 * d2 * d2
                d8 = d6 * d2
                sw = _smoother(d)
                ev_v[g, pl.ds(s, 16)] = -0.5 * sw * (s6 * c6 / (d6 + tmp6) +
                                                     s8 * c8 / (d8 + tmp8))

            if g + 4 < R2:
                dsc[g + 4] = pltpu.async_copy(
                    table_h.at[cls_v.at[g + 4]],
                    rows_v.at[pl.ds(((g + 4) % 4) * 128, 128)],
                    sems[(g + 4) % 4])

        for g in range(R2):
            pltpu.sync_copy(ev_v.at[g], acc.at[ii_v.at[g]], add=True)

    plsc.subcore_barrier()
    pltpu.sync_copy(acc.at[pl.ds(sid * ASL, ASL)], zbuf)
    pltpu.sync_copy(zbuf, epart.at[pl.ds(cid * NAP + sid * ASL, ASL)])


def _combine_body(a_ref, o_ref):
    o_ref[...] = a_ref[0] + a_ref[1]


def _combine(parts):
    out = pl.pallas_call(
        _combine_body,
        out_shape=jax.ShapeDtypeStruct((NAP // 128, 128), jnp.float32),
    )(parts.reshape(2, NAP // 128, 128))
    return out.reshape(NAP)


@jax.jit
def kernel(atomic_numbers, distances, idx_i, idx_j, c6ab, rcov, r2r4,
           d3_s6, d3_s8, d3_a1, d3_a2):
    f32 = jnp.float32
    i32 = jnp.int32
    pad_p = NPP - N_PAIRS
    ii = jnp.pad(idx_i.astype(i32), (0, pad_p)).reshape(PROWS, 128)
    jj = jnp.pad(idx_j.astype(i32), (0, pad_p)).reshape(PROWS, 128)
    d = jnp.pad(distances.astype(f32), (0, pad_p),
                constant_values=11.0).reshape(PROWS, 128)
    atoms = jnp.pad(atomic_numbers.astype(i32), (0, NAP - N_ATOMS))
    rcov96 = jnp.pad(rcov.astype(f32), (0, 96 - N_ELEM))
    u96 = jnp.pad((3.0 ** 0.25) * jnp.sqrt(r2r4.astype(f32)),
                  (0, 96 - N_ELEM))
    table = c6ab.astype(f32).transpose(0, 1, 4, 2, 3).reshape(N_ELEM * N_ELEM, 75)
    table = jnp.pad(table, ((0, 0), (0, TROW - 75)))
    params = jnp.pad(jnp.stack([d3_s6, d3_s8, d3_a1, d3_a2]).astype(f32),
                     (0, 12))

    mesh = plsc.VectorSubcoreMesh(core_axis_name="c", subcore_axis_name="s",
                                  num_cores=NC, num_subcores=NS)
    cparams = pltpu.CompilerParams(needs_layout_passes=False,
                                   use_tc_tiling_on_sc=False)

    p1 = pl.kernel(
        _phase1_body,
        out_type=[
            jax.ShapeDtypeStruct((NC * NAP,), f32),
            jax.ShapeDtypeStruct((PROWS, 128), i32),
            jax.ShapeDtypeStruct((PROWS, 128), f32),
        ],
        mesh=mesh,
        compiler_params=cparams,
        scratch_types=[
            pltpu.VMEM((NAP,), i32),
            pltpu.VMEM((96,), f32),
            pltpu.VMEM((96,), f32),
            pltpu.VMEM((R1, 128), i32),
            pltpu.VMEM((R1, 128), i32),
            pltpu.VMEM((R1, 128), f32),
            pltpu.VMEM((R1, 128), f32),
            pltpu.VMEM((R1, 128), i32),
            pltpu.VMEM((R1, 128), f32),
            pltpu.VMEM((ASL,), f32),
            pltpu.VMEM_SHARED((NAP,), f32),
        ],
    )
    ncpart, cls, srr = p1(atoms, ii, jj, d, rcov96, u96)
    nc = _combine(ncpart)

    p2 = pl.kernel(
        _phase2_body,
        out_type=jax.ShapeDtypeStruct((NC * NAP,), f32),
        mesh=mesh,
        compiler_params=cparams,
        scratch_types=[
            pltpu.VMEM((NAP,), f32),
            pltpu.VMEM((16,), f32),
            pltpu.VMEM((R2, 128), i32),
            pltpu.VMEM((R2, 128), i32),
            pltpu.VMEM((R2, 128), f32),
            pltpu.VMEM((R2, 128), f32),
            pltpu.VMEM((R2, 128), f32),
            pltpu.VMEM((R2, 128), i32),
            pltpu.VMEM((512, TROW), f32),
            pltpu.VMEM((ASL,), f32),
            pltpu.SemaphoreType.DMA,
            pltpu.SemaphoreType.DMA,
            pltpu.SemaphoreType.DMA,
            pltpu.SemaphoreType.DMA,
            pltpu.VMEM_SHARED((NAP,), f32),
        ],
    )
    epart = p2(table, nc, ii, jj, d, srr, cls, params)
    return _combine(epart)[:N_ATOMS]


# speedup vs baseline: 466.2963x; 1.3001x over previous
import jax
import jax.numpy as jnp
from jax import lax
from jax.experimental import pallas as pl
from jax.experimental.pallas import tpu as pltpu
from jax.experimental.pallas import tpu_sc as plsc

N_ATOMS = 50000
N_PAIRS = 1600000
N_ELEM = 95
CUTOFF = 10.0
WIDTH = 2.0
CUTON = CUTOFF - WIDTH
K1 = 16.0
K3 = -4.0

NC = 2
NS = 16
NT = NC * NS
NAP = 50176
ASL = NAP // NS
PT = 50176
NPP = NT * PT
PROWS = NPP // 128

R1 = 8
C1 = R1 * 128
R2 = 8
C2 = R2 * 128
TROW = 56


def _smoother(d):
    x = (CUTOFF - d) * (1.0 / WIDTH)
    poly = ((6.0 * x - 15.0) * x + 10.0) * x * x * x
    return jnp.where(d < CUTON, 1.0, jnp.where(d >= CUTOFF, 0.0, poly))


def _phase1_body(atoms_h, ii_h, jj_h, d_h, rcov_h, u_h,
                 ncpart, cls_o, srr_o,
                 atoms_v, rcov_v, u_v, ii_v, jj_v, d_v,
                 vals_v, cls_v, srr_v, zbuf, acc):
    cid = lax.axis_index("c")
    sid = lax.axis_index("s")
    wid = cid * NS + sid

    pltpu.sync_copy(atoms_h, atoms_v)
    pltpu.sync_copy(rcov_h, rcov_v)
    pltpu.sync_copy(u_h, u_v)

    @pl.loop(0, ASL // 16)
    def _z(k):
        zbuf[pl.ds(k * 16, 16)] = jnp.zeros((16,), jnp.float32)

    pltpu.sync_copy(zbuf, acc.at[pl.ds(sid * ASL, ASL)])
    plsc.subcore_barrier()

    row_base = wid * (PT // 128)

    @pl.loop(0, PT // C1)
    def _chunk(c):
        roff = row_base + c * R1
        pltpu.sync_copy(ii_h.at[pl.ds(roff, R1)], ii_v)
        pltpu.sync_copy(jj_h.at[pl.ds(roff, R1)], jj_v)
        pltpu.sync_copy(d_h.at[pl.ds(roff, R1)], d_v)

        for g in range(R1):
            @pl.loop(0, 8)
            def _grp(k):
                s = k * 16
                ii = ii_v[g, pl.ds(s, 16)]
                jj = jj_v[g, pl.ds(s, 16)]
                d = d_v[g, pl.ds(s, 16)]
                zi = plsc.load_gather(atoms_v, [ii])
                zj = plsc.load_gather(atoms_v, [jj])
                rco = (plsc.load_gather(rcov_v, [zi]) +
                       plsc.load_gather(rcov_v, [zj]))
                damp = 1.0 / (1.0 + jnp.exp(-K1 * (rco / d - 1.0)))
                vals_v[g, pl.ds(s, 16)] = damp * _smoother(d)
                cls_v[g, pl.ds(s, 16)] = zi * N_ELEM + zj
                srr_v[g, pl.ds(s, 16)] = (plsc.load_gather(u_v, [zi]) *
                                          plsc.load_gather(u_v, [zj]))

        for g in range(R1):
            pltpu.sync_copy(vals_v.at[g], acc.at[ii_v.at[g]], add=True)
        pltpu.sync_copy(cls_v, cls_o.at[pl.ds(roff, R1)])
        pltpu.sync_copy(srr_v, srr_o.at[pl.ds(roff, R1)])

    plsc.subcore_barrier()
    pltpu.sync_copy(acc.at[pl.ds(sid * ASL, ASL)], zbuf)
    pltpu.sync_copy(zbuf, ncpart.at[pl.ds(cid * NAP + sid * ASL, ASL)])


def _phase2_body(table_h, nc_h, ii_h, jj_h, d_h, srr_h, cls_h, params_h,
                 epart,
                 nc_v, params_v, ii_v, jj_v, d_v, srr_v, ev_v,
                 cls_v, rows_v, zbuf, sem0, sem1, sem2, sem3, acc):
    cid = lax.axis_index("c")
    sid = lax.axis_index("s")
    wid = cid * NS + sid

    pltpu.sync_copy(nc_h, nc_v)
    pltpu.sync_copy(params_h, params_v)

    @pl.loop(0, ASL // 16)
    def _z(k):
        zbuf[pl.ds(k * 16, 16)] = jnp.zeros((16,), jnp.float32)

    pltpu.sync_copy(zbuf, acc.at[pl.ds(sid * ASL, ASL)])
    plsc.subcore_barrier()

    pv = params_v[pl.ds(0, 16)]
    s6 = pv[0]
    s8 = pv[1]
    a1 = pv[2]
    a2 = pv[3]
    row_base = wid * (PT // 128)
    iota = lax.iota(jnp.int32, 16)

    @pl.loop(0, PT // C2)
    def _chunk(c):
        roff = row_base + c * R2
        pltpu.sync_copy(cls_h.at[pl.ds(roff, R2)], cls_v)
        pltpu.sync_copy(ii_h.at[pl.ds(roff, R2)], ii_v)
        pltpu.sync_copy(jj_h.at[pl.ds(roff, R2)], jj_v)
        pltpu.sync_copy(d_h.at[pl.ds(roff, R2)], d_v)
        pltpu.sync_copy(srr_h.at[pl.ds(roff, R2)], srr_v)
        sems = [sem0, sem1, sem2, sem3]
        dsc = [None] * R2
        for g in range(4):
            dsc[g] = pltpu.async_copy(table_h.at[cls_v.at[g]],
                                      rows_v.at[pl.ds((g % 4) * 128, 128)],
                                      sems[g % 4])
        for g in range(R2):
            dsc[g].wait()

            @pl.loop(0, 8)
            def _grp(k):
                s = k * 16
                ii = ii_v[g, pl.ds(s, 16)]
                jj = jj_v[g, pl.ds(s, 16)]
                d = d_v[g, pl.ds(s, 16)]
                srr = srr_v[g, pl.ds(s, 16)]
                nci = plsc.load_gather(nc_v, [ii])
                ncj = plsc.load_gather(nc_v, [jj])
                pr = (g % 4) * 128 + s + iota
                rsum = jnp.zeros((16,), jnp.float32)
                csum = jnp.zeros((16,), jnp.float32)
                for t in range(25):
                    c0 = plsc.bitcast(
                        plsc.load_gather(rows_v, [pr, jnp.full((16,), t, jnp.int32)]),
                        jnp.float32)
                    w12 = plsc.load_gather(rows_v, [pr, jnp.full((16,), 25 + t, jnp.int32)])
                    c1 = plsc.bitcast(lax.shift_left(w12, 16), jnp.float32)
                    c2 = plsc.bitcast(w12 & jnp.int32(-65536), jnp.float32)
                    dr1 = c1 - nci
                    dr2 = c2 - ncj
                    w = jnp.exp(K3 * (dr1 * dr1 + dr2 * dr2))
                    m = c0 > 0.0
                    rsum = rsum + jnp.where(m, w, 0.0)
                    csum = csum + jnp.where(m, w * c0, 0.0)
                c6 = jnp.where(rsum > 0.0, csum / rsum, -1e38)
                c8 = c6 * srr * srr
                tmp = a1 * srr + a2
                tmp2 = tmp * tmp
                tmp6 = tmp2 * tmp2 * tmp2
                tmp8 = tmp6 * tmp2
                d2 = d * d
                d6 = d2 * d2 * d2
                d8 = d6 * d2
                sw = _smoother(d)
                ev_v[g, pl.ds(s, 16)] = -0.5 * sw * (s6 * c6 / (d6 + tmp6) +
                                                     s8 * c8 / (d8 + tmp8))

            if g + 4 < R2:
                dsc[g + 4] = pltpu.async_copy(
                    table_h.at[cls_v.at[g + 4]],
                    rows_v.at[pl.ds(((g + 4) % 4) * 128, 128)],
                    sems[(g + 4) % 4])

        for g in range(R2):
            pltpu.sync_copy(ev_v.at[g], acc.at[ii_v.at[g]], add=True)

    plsc.subcore_barrier()
    pltpu.sync_copy(acc.at[pl.ds(sid * ASL, ASL)], zbuf)
    pltpu.sync_copy(zbuf, epart.at[pl.ds(cid * NAP + sid * ASL, ASL)])


def _combine_body(a_ref, o_ref):
    o_ref[...] = a_ref[0] + a_ref[1]


def _combine(parts):
    out = pl.pallas_call(
        _combine_body,
        out_shape=jax.ShapeDtypeStruct((NAP // 128, 128), jnp.float32),
    )(parts.reshape(2, NAP // 128, 128))
    return out.reshape(NAP)


@jax.jit
def kernel(atomic_numbers, distances, idx_i, idx_j, c6ab, rcov, r2r4,
           d3_s6, d3_s8, d3_a1, d3_a2):
    f32 = jnp.float32
    i32 = jnp.int32
    pad_p = NPP - N_PAIRS
    ii = jnp.pad(idx_i.astype(i32), (0, pad_p)).reshape(PROWS, 128)
    jj = jnp.pad(idx_j.astype(i32), (0, pad_p)).reshape(PROWS, 128)
    d = jnp.pad(distances.astype(f32), (0, pad_p),
                constant_values=11.0).reshape(PROWS, 128)
    atoms = jnp.pad(atomic_numbers.astype(i32), (0, NAP - N_ATOMS))
    rcov96 = jnp.pad(rcov.astype(f32), (0, 96 - N_ELEM))
    u96 = jnp.pad((3.0 ** 0.25) * jnp.sqrt(r2r4.astype(f32)),
                  (0, 96 - N_ELEM))
    c6f = c6ab.astype(f32)
    cn0 = c6f[..., 0].reshape(N_ELEM * N_ELEM, 25)
    u32 = jnp.uint32
    w1 = lax.bitcast_convert_type(c6f[..., 1].astype(jnp.bfloat16), jnp.uint16
                                  ).reshape(N_ELEM * N_ELEM, 25).astype(u32)
    w2 = lax.bitcast_convert_type(c6f[..., 2].astype(jnp.bfloat16), jnp.uint16
                                  ).reshape(N_ELEM * N_ELEM, 25).astype(u32)
    packed = lax.bitcast_convert_type(w1 | (w2 << 16), i32)
    table = jnp.concatenate(
        [lax.bitcast_convert_type(cn0, i32), packed,
         jnp.zeros((N_ELEM * N_ELEM, TROW - 50), i32)], axis=1)
    params = jnp.pad(jnp.stack([d3_s6, d3_s8, d3_a1, d3_a2]).astype(f32),
                     (0, 12))

    mesh = plsc.VectorSubcoreMesh(core_axis_name="c", subcore_axis_name="s",
                                  num_cores=NC, num_subcores=NS)
    cparams = pltpu.CompilerParams(needs_layout_passes=False,
                                   use_tc_tiling_on_sc=False)

    p1 = pl.kernel(
        _phase1_body,
        out_type=[
            jax.ShapeDtypeStruct((NC * NAP,), f32),
            jax.ShapeDtypeStruct((PROWS, 128), i32),
            jax.ShapeDtypeStruct((PROWS, 128), f32),
        ],
        mesh=mesh,
        compiler_params=cparams,
        scratch_types=[
            pltpu.VMEM((NAP,), i32),
            pltpu.VMEM((96,), f32),
            pltpu.VMEM((96,), f32),
            pltpu.VMEM((R1, 128), i32),
            pltpu.VMEM((R1, 128), i32),
            pltpu.VMEM((R1, 128), f32),
            pltpu.VMEM((R1, 128), f32),
            pltpu.VMEM((R1, 128), i32),
            pltpu.VMEM((R1, 128), f32),
            pltpu.VMEM((ASL,), f32),
            pltpu.VMEM_SHARED((NAP,), f32),
        ],
    )
    ncpart, cls, srr = p1(atoms, ii, jj, d, rcov96, u96)
    nc = _combine(ncpart)

    p2 = pl.kernel(
        _phase2_body,
        out_type=jax.ShapeDtypeStruct((NC * NAP,), f32),
        mesh=mesh,
        compiler_params=cparams,
        scratch_types=[
            pltpu.VMEM((NAP,), f32),
            pltpu.VMEM((16,), f32),
            pltpu.VMEM((R2, 128), i32),
            pltpu.VMEM((R2, 128), i32),
            pltpu.VMEM((R2, 128), f32),
            pltpu.VMEM((R2, 128), f32),
            pltpu.VMEM((R2, 128), f32),
            pltpu.VMEM((R2, 128), i32),
            pltpu.VMEM((512, TROW), i32),
            pltpu.VMEM((ASL,), f32),
            pltpu.SemaphoreType.DMA,
            pltpu.SemaphoreType.DMA,
            pltpu.SemaphoreType.DMA,
            pltpu.SemaphoreType.DMA,
            pltpu.VMEM_SHARED((NAP,), f32),
        ],
    )
    epart = p2(table, nc, ii, jj, d, srr, cls, params)
    return _combine(epart)[:N_ATOMS]


# speedup vs baseline: 485.5901x; 1.0414x over previous
import jax
import jax.numpy as jnp
from jax import lax
from jax.experimental import pallas as pl
from jax.experimental.pallas import tpu as pltpu
from jax.experimental.pallas import tpu_sc as plsc

N_ATOMS = 50000
N_PAIRS = 1600000
N_ELEM = 95
CUTOFF = 10.0
WIDTH = 2.0
CUTON = CUTOFF - WIDTH
K1 = 16.0
K3 = -4.0

NC = 2
NS = 16
NT = NC * NS
NAP = 50176
ASL = NAP // NS
PT = 50176
NPP = NT * PT
PROWS = NPP // 128

R1 = 8
C1 = R1 * 128
R2 = 8
C2 = R2 * 128
TROW = 56


def _smoother(d):
    x = (CUTOFF - d) * (1.0 / WIDTH)
    poly = ((6.0 * x - 15.0) * x + 10.0) * x * x * x
    return jnp.where(d < CUTON, 1.0, jnp.where(d >= CUTOFF, 0.0, poly))


def _phase1_body(atoms_h, ii_h, jj_h, d_h, rcov_h, u_h,
                 ncpart, cls_o, srr_o,
                 atoms_v, rcov_v, u_v, ii_v, jj_v, d_v,
                 ii1_v, vals1_v, cls_v, srr_v, zbuf, acc):
    cid = lax.axis_index("c")
    sid = lax.axis_index("s")
    wid = cid * NS + sid

    pltpu.sync_copy(atoms_h, atoms_v)
    pltpu.sync_copy(rcov_h, rcov_v)
    pltpu.sync_copy(u_h, u_v)

    @pl.loop(0, ASL // 16)
    def _z(k):
        zbuf[pl.ds(k * 16, 16)] = jnp.zeros((16,), jnp.float32)

    pltpu.sync_copy(zbuf, acc.at[pl.ds(sid * ASL, ASL)])
    plsc.subcore_barrier()

    row_base = wid * (PT // 128)

    @pl.loop(0, PT // C1)
    def _chunk(c):
        roff = row_base + c * R1
        pltpu.sync_copy(ii_h.at[pl.ds(roff, R1)], ii_v)
        pltpu.sync_copy(jj_h.at[pl.ds(roff, R1)], jj_v)
        pltpu.sync_copy(d_h.at[pl.ds(roff, R1)], d_v)

        for g in range(R1):
            @pl.loop(0, 8)
            def _grp(k):
                s = k * 16
                ii = ii_v[g, pl.ds(s, 16)]
                jj = jj_v[g, pl.ds(s, 16)]
                d = d_v[g, pl.ds(s, 16)]
                zi = plsc.load_gather(atoms_v, [ii])
                zj = plsc.load_gather(atoms_v, [jj])
                rco = (plsc.load_gather(rcov_v, [zi]) +
                       plsc.load_gather(rcov_v, [zj]))
                damp = 1.0 / (1.0 + jnp.exp(-K1 * (rco / d - 1.0)))
                ii1_v[pl.ds(g * 128 + s, 16)] = ii
                vals1_v[pl.ds(g * 128 + s, 16)] = damp * _smoother(d)
                cls_v[g, pl.ds(s, 16)] = zi * N_ELEM + zj
                srr_v[g, pl.ds(s, 16)] = (plsc.load_gather(u_v, [zi]) *
                                          plsc.load_gather(u_v, [zj]))

        pltpu.sync_copy(vals1_v, acc.at[ii1_v], add=True)
        pltpu.sync_copy(cls_v, cls_o.at[pl.ds(roff, R1)])
        pltpu.sync_copy(srr_v, srr_o.at[pl.ds(roff, R1)])

    plsc.subcore_barrier()
    pltpu.sync_copy(acc.at[pl.ds(sid * ASL, ASL)], zbuf)
    pltpu.sync_copy(zbuf, ncpart.at[pl.ds(cid * NAP + sid * ASL, ASL)])


def _phase2_body(table_h, nc_h, ii_h, jj_h, d_h, srr_h, cls_h, params_h,
                 epart,
                 nc_v, params_v, ii_v, jj_v, d_v, srr_v, ii1_v, ev1_v,
                 cls_v, rows_v, zbuf, sem0, sem1, sem2, sem3, acc):
    cid = lax.axis_index("c")
    sid = lax.axis_index("s")
    wid = cid * NS + sid

    pltpu.sync_copy(nc_h, nc_v)
    pltpu.sync_copy(params_h, params_v)

    @pl.loop(0, ASL // 16)
    def _z(k):
        zbuf[pl.ds(k * 16, 16)] = jnp.zeros((16,), jnp.float32)

    pltpu.sync_copy(zbuf, acc.at[pl.ds(sid * ASL, ASL)])
    plsc.subcore_barrier()

    pv = params_v[pl.ds(0, 16)]
    s6 = pv[0]
    s8 = pv[1]
    a1 = pv[2]
    a2 = pv[3]
    row_base = wid * (PT // 128)
    iota = lax.iota(jnp.int32, 16)

    @pl.loop(0, PT // C2)
    def _chunk(c):
        roff = row_base + c * R2
        pltpu.sync_copy(cls_h.at[pl.ds(roff, R2)], cls_v)
        pltpu.sync_copy(ii_h.at[pl.ds(roff, R2)], ii_v)
        pltpu.sync_copy(jj_h.at[pl.ds(roff, R2)], jj_v)
        pltpu.sync_copy(d_h.at[pl.ds(roff, R2)], d_v)
        pltpu.sync_copy(srr_h.at[pl.ds(roff, R2)], srr_v)
        sems = [sem0, sem1, sem2, sem3]
        dsc = [None] * R2
        for g in range(4):
            dsc[g] = pltpu.async_copy(table_h.at[cls_v.at[g]],
                                      rows_v.at[pl.ds((g % 4) * 128, 128)],
                                      sems[g % 4])
        for g in range(R2):
            dsc[g].wait()

            @pl.loop(0, 8)
            def _grp(k):
                s = k * 16
                ii = ii_v[g, pl.ds(s, 16)]
                jj = jj_v[g, pl.ds(s, 16)]
                d = d_v[g, pl.ds(s, 16)]
                srr = srr_v[g, pl.ds(s, 16)]
                ii1_v[pl.ds(g * 128 + s, 16)] = ii
                nci = plsc.load_gather(nc_v, [ii])
                ncj = plsc.load_gather(nc_v, [jj])
                pr = (g % 4) * 128 + s + iota
                rsum = jnp.zeros((16,), jnp.float32)
                csum = jnp.zeros((16,), jnp.float32)
                for t in range(25):
                    c0 = plsc.bitcast(
                        plsc.load_gather(rows_v, [pr, jnp.full((16,), t, jnp.int32)]),
                        jnp.float32)
                    w12 = plsc.load_gather(rows_v, [pr, jnp.full((16,), 25 + t, jnp.int32)])
                    c1 = plsc.bitcast(lax.shift_left(w12, 16), jnp.float32)
                    c2 = plsc.bitcast(w12 & jnp.int32(-65536), jnp.float32)
                    dr1 = c1 - nci
                    dr2 = c2 - ncj
                    w = jnp.exp(K3 * (dr1 * dr1 + dr2 * dr2))
                    m = c0 > 0.0
                    rsum = rsum + jnp.where(m, w, 0.0)
                    csum = csum + jnp.where(m, w * c0, 0.0)
                c6 = jnp.where(rsum > 0.0, csum / rsum, -1e38)
                c8 = c6 * srr * srr
                tmp = a1 * srr + a2
                tmp2 = tmp * tmp
                tmp6 = tmp2 * tmp2 * tmp2
                tmp8 = tmp6 * tmp2
                d2 = d * d
                d6 = d2 * d2 * d2
                d8 = d6 * d2
                sw = _smoother(d)
                ev1_v[pl.ds(g * 128 + s, 16)] = -0.5 * sw * (s6 * c6 / (d6 + tmp6) +
                                                             s8 * c8 / (d8 + tmp8))

            if g + 4 < R2:
                dsc[g + 4] = pltpu.async_copy(
                    table_h.at[cls_v.at[g + 4]],
                    rows_v.at[pl.ds(((g + 4) % 4) * 128, 128)],
                    sems[(g + 4) % 4])

        pltpu.sync_copy(ev1_v, acc.at[ii1_v], add=True)

    plsc.subcore_barrier()
    pltpu.sync_copy(acc.at[pl.ds(sid * ASL, ASL)], zbuf)
    pltpu.sync_copy(zbuf, epart.at[pl.ds(cid * NAP + sid * ASL, ASL)])


def _combine_body(a_ref, o_ref):
    o_ref[...] = a_ref[0] + a_ref[1]


def _combine(parts):
    out = pl.pallas_call(
        _combine_body,
        out_shape=jax.ShapeDtypeStruct((NAP // 128, 128), jnp.float32),
    )(parts.reshape(2, NAP // 128, 128))
    return out.reshape(NAP)


@jax.jit
def kernel(atomic_numbers, distances, idx_i, idx_j, c6ab, rcov, r2r4,
           d3_s6, d3_s8, d3_a1, d3_a2):
    f32 = jnp.float32
    i32 = jnp.int32
    pad_p = NPP - N_PAIRS
    ii = jnp.pad(idx_i.astype(i32), (0, pad_p)).reshape(PROWS, 128)
    jj = jnp.pad(idx_j.astype(i32), (0, pad_p)).reshape(PROWS, 128)
    d = jnp.pad(distances.astype(f32), (0, pad_p),
                constant_values=11.0).reshape(PROWS, 128)
    atoms = jnp.pad(atomic_numbers.astype(i32), (0, NAP - N_ATOMS))
    rcov96 = jnp.pad(rcov.astype(f32), (0, 96 - N_ELEM))
    u96 = jnp.pad((3.0 ** 0.25) * jnp.sqrt(r2r4.astype(f32)),
                  (0, 96 - N_ELEM))
    c6f = c6ab.astype(f32)
    cn0 = c6f[..., 0].reshape(N_ELEM * N_ELEM, 25)
    u32 = jnp.uint32
    w1 = lax.bitcast_convert_type(c6f[..., 1].astype(jnp.bfloat16), jnp.uint16
                                  ).reshape(N_ELEM * N_ELEM, 25).astype(u32)
    w2 = lax.bitcast_convert_type(c6f[..., 2].astype(jnp.bfloat16), jnp.uint16
                                  ).reshape(N_ELEM * N_ELEM, 25).astype(u32)
    packed = lax.bitcast_convert_type(w1 | (w2 << 16), i32)
    table = jnp.concatenate(
        [lax.bitcast_convert_type(cn0, i32), packed,
         jnp.zeros((N_ELEM * N_ELEM, TROW - 50), i32)], axis=1)
    params = jnp.pad(jnp.stack([d3_s6, d3_s8, d3_a1, d3_a2]).astype(f32),
                     (0, 12))

    mesh = plsc.VectorSubcoreMesh(core_axis_name="c", subcore_axis_name="s",
                                  num_cores=NC, num_subcores=NS)
    cparams = pltpu.CompilerParams(needs_layout_passes=False,
                                   use_tc_tiling_on_sc=False)

    p1 = pl.kernel(
        _phase1_body,
        out_type=[
            jax.ShapeDtypeStruct((NC * NAP,), f32),
            jax.ShapeDtypeStruct((PROWS, 128), i32),
            jax.ShapeDtypeStruct((PROWS, 128), f32),
        ],
        mesh=mesh,
        compiler_params=cparams,
        scratch_types=[
            pltpu.VMEM((NAP,), i32),
            pltpu.VMEM((96,), f32),
            pltpu.VMEM((96,), f32),
            pltpu.VMEM((R1, 128), i32),
            pltpu.VMEM((R1, 128), i32),
            pltpu.VMEM((R1, 128), f32),
            pltpu.VMEM((C1,), i32),
            pltpu.VMEM((C1,), f32),
            pltpu.VMEM((R1, 128), i32),
            pltpu.VMEM((R1, 128), f32),
            pltpu.VMEM((ASL,), f32),
            pltpu.VMEM_SHARED((NAP,), f32),
        ],
    )
    ncpart, cls, srr = p1(atoms, ii, jj, d, rcov96, u96)
    nc = _combine(ncpart)

    p2 = pl.kernel(
        _phase2_body,
        out_type=jax.ShapeDtypeStruct((NC * NAP,), f32),
        mesh=mesh,
        compiler_params=cparams,
        scratch_types=[
            pltpu.VMEM((NAP,), f32),
            pltpu.VMEM((16,), f32),
            pltpu.VMEM((R2, 128), i32),
            pltpu.VMEM((R2, 128), i32),
            pltpu.VMEM((R2, 128), f32),
            pltpu.VMEM((R2, 128), f32),
            pltpu.VMEM((C2,), i32),
            pltpu.VMEM((C2,), f32),
            pltpu.VMEM((R2, 128), i32),
            pltpu.VMEM((512, TROW), i32),
            pltpu.VMEM((ASL,), f32),
            pltpu.SemaphoreType.DMA,
            pltpu.SemaphoreType.DMA,
            pltpu.SemaphoreType.DMA,
            pltpu.SemaphoreType.DMA,
            pltpu.VMEM_SHARED((NAP,), f32),
        ],
    )
    epart = p2(table, nc, ii, jj, d, srr, cls, params)
    return _combine(epart)[:N_ATOMS]


# speedup vs baseline: 538.4111x; 1.1088x over previous
import jax
import jax.numpy as jnp
from jax import lax
from jax.experimental import pallas as pl
from jax.experimental.pallas import tpu as pltpu
from jax.experimental.pallas import tpu_sc as plsc

N_ATOMS = 50000
N_PAIRS = 1600000
N_ELEM = 95
CUTOFF = 10.0
WIDTH = 2.0
CUTON = CUTOFF - WIDTH
K1 = 16.0
K3 = -4.0

NC = 2
NS = 16
NT = NC * NS
NAP = 50176
ASL = NAP // NS
PT = 50176
NPP = NT * PT
PROWS = NPP // 128

R1 = 8
C1 = R1 * 128
R2 = 8
C2 = R2 * 128
TROW = 56
NTROWS = 9216


def _smoother(d):
    x = (CUTOFF - d) * (1.0 / WIDTH)
    poly = ((6.0 * x - 15.0) * x + 10.0) * x * x * x
    return jnp.where(d < CUTON, 1.0, jnp.where(d >= CUTOFF, 0.0, poly))


def _phase1_body(atoms_h, ii_h, jj_h, d_h, rcov_h, u_h,
                 ncpart, cls_o, srr_o,
                 atoms_v, rcov_v, u_v, ii_v, jj_v, d_v,
                 ii1_v, vals1_v, cls_v, srr_v, zbuf, acc):
    cid = lax.axis_index("c")
    sid = lax.axis_index("s")
    wid = cid * NS + sid

    pltpu.sync_copy(atoms_h, atoms_v)
    pltpu.sync_copy(rcov_h, rcov_v)
    pltpu.sync_copy(u_h, u_v)

    @pl.loop(0, ASL // 16)
    def _z(k):
        zbuf[pl.ds(k * 16, 16)] = jnp.zeros((16,), jnp.float32)

    pltpu.sync_copy(zbuf, acc.at[pl.ds(sid * ASL, ASL)])
    plsc.subcore_barrier()

    row_base = wid * (PT // 128)

    @pl.loop(0, PT // C1)
    def _chunk(c):
        roff = row_base + c * R1
        pltpu.sync_copy(ii_h.at[pl.ds(roff, R1)], ii_v)
        pltpu.sync_copy(jj_h.at[pl.ds(roff, R1)], jj_v)
        pltpu.sync_copy(d_h.at[pl.ds(roff, R1)], d_v)

        for g in range(R1):
            @pl.loop(0, 8)
            def _grp(k):
                s = k * 16
                ii = ii_v[g, pl.ds(s, 16)]
                jj = jj_v[g, pl.ds(s, 16)]
                d = d_v[g, pl.ds(s, 16)]
                zi = plsc.load_gather(atoms_v, [ii])
                zj = plsc.load_gather(atoms_v, [jj])
                rco = (plsc.load_gather(rcov_v, [zi]) +
                       plsc.load_gather(rcov_v, [zj]))
                damp = 1.0 / (1.0 + jnp.exp(-K1 * (rco / d - 1.0)))
                ii1_v[pl.ds(g * 128 + s, 16)] = ii
                vals1_v[pl.ds(g * 128 + s, 16)] = damp * _smoother(d)
                cls_v[g, pl.ds(s, 16)] = zi * N_ELEM + zj
                srr_v[g, pl.ds(s, 16)] = (plsc.load_gather(u_v, [zi]) *
                                          plsc.load_gather(u_v, [zj]))

        pltpu.sync_copy(vals1_v, acc.at[ii1_v], add=True)
        pltpu.sync_copy(cls_v, cls_o.at[pl.ds(roff, R1)])
        pltpu.sync_copy(srr_v, srr_o.at[pl.ds(roff, R1)])

    plsc.subcore_barrier()
    pltpu.sync_copy(acc.at[pl.ds(sid * ASL, ASL)], zbuf)
    pltpu.sync_copy(zbuf, ncpart.at[pl.ds(cid * NAP + sid * ASL, ASL)])


def _phase2_body(table_h, nc_h, ii_h, jj_h, d_h, srr_h, cls_h, params_h,
                 epart,
                 nc_v, params_v, ii_v, jj_v, d_v, srr_v, ii1_v, ev1_v,
                 cls_v, rows_v, zbuf, sem0, sem1, sem2, sem3, spm_table, acc):
    cid = lax.axis_index("c")
    sid = lax.axis_index("s")
    wid = cid * NS + sid

    pltpu.sync_copy(nc_h, nc_v)
    pltpu.sync_copy(params_h, params_v)

    @pl.loop(0, ASL // 16)
    def _z(k):
        zbuf[pl.ds(k * 16, 16)] = jnp.zeros((16,), jnp.float32)

    pltpu.sync_copy(zbuf, acc.at[pl.ds(sid * ASL, ASL)])

    tb = sid * (NTROWS // NS)

    @pl.loop(0, NTROWS // NS // 64)
    def _stage(j):
        pltpu.sync_copy(table_h.at[pl.ds(tb + j * 64, 64)],
                        rows_v.at[pl.ds(0, 64)])
        pltpu.sync_copy(rows_v.at[pl.ds(0, 64)],
                        spm_table.at[pl.ds(tb + j * 64, 64)])

    plsc.subcore_barrier()

    pv = params_v[pl.ds(0, 16)]
    s6 = pv[0]
    s8 = pv[1]
    a1 = pv[2]
    a2 = pv[3]
    row_base = wid * (PT // 128)
    iota = lax.iota(jnp.int32, 16)

    @pl.loop(0, PT // C2)
    def _chunk(c):
        roff = row_base + c * R2
        pltpu.sync_copy(cls_h.at[pl.ds(roff, R2)], cls_v)
        pltpu.sync_copy(ii_h.at[pl.ds(roff, R2)], ii_v)
        pltpu.sync_copy(jj_h.at[pl.ds(roff, R2)], jj_v)
        pltpu.sync_copy(d_h.at[pl.ds(roff, R2)], d_v)
        pltpu.sync_copy(srr_h.at[pl.ds(roff, R2)], srr_v)
        sems = [sem0, sem1, sem2, sem3]
        dsc = [None] * R2
        for g in range(4):
            dsc[g] = pltpu.async_copy(spm_table.at[cls_v.at[g]],
                                      rows_v.at[pl.ds((g % 4) * 128, 128)],
                                      sems[g % 4])
        for g in range(R2):
            dsc[g].wait()

            @pl.loop(0, 8)
            def _grp(k):
                s = k * 16
                ii = ii_v[g, pl.ds(s, 16)]
                jj = jj_v[g, pl.ds(s, 16)]
                d = d_v[g, pl.ds(s, 16)]
                srr = srr_v[g, pl.ds(s, 16)]
                ii1_v[pl.ds(g * 128 + s, 16)] = ii
                nci = plsc.load_gather(nc_v, [ii])
                ncj = plsc.load_gather(nc_v, [jj])
                pr = (g % 4) * 128 + s + iota
                rsum = jnp.zeros((16,), jnp.float32)
                csum = jnp.zeros((16,), jnp.float32)
                for t in range(25):
                    c0 = plsc.bitcast(
                        plsc.load_gather(rows_v, [pr, jnp.full((16,), t, jnp.int32)]),
                        jnp.float32)
                    w12 = plsc.load_gather(rows_v, [pr, jnp.full((16,), 25 + t, jnp.int32)])
                    c1 = plsc.bitcast(lax.shift_left(w12, 16), jnp.float32)
                    c2 = plsc.bitcast(w12 & jnp.int32(-65536), jnp.float32)
                    dr1 = c1 - nci
                    dr2 = c2 - ncj
                    w = jnp.exp(K3 * (dr1 * dr1 + dr2 * dr2))
                    m = c0 > 0.0
                    rsum = rsum + jnp.where(m, w, 0.0)
                    csum = csum + jnp.where(m, w * c0, 0.0)
                c6 = jnp.where(rsum > 0.0, csum / rsum, -1e38)
                c8 = c6 * srr * srr
                tmp = a1 * srr + a2
                tmp2 = tmp * tmp
                tmp6 = tmp2 * tmp2 * tmp2
                tmp8 = tmp6 * tmp2
                d2 = d * d
                d6 = d2 * d2 * d2
                d8 = d6 * d2
                sw = _smoother(d)
                ev1_v[pl.ds(g * 128 + s, 16)] = -0.5 * sw * (s6 * c6 / (d6 + tmp6) +
                                                             s8 * c8 / (d8 + tmp8))

            if g + 4 < R2:
                dsc[g + 4] = pltpu.async_copy(
                    spm_table.at[cls_v.at[g + 4]],
                    rows_v.at[pl.ds(((g + 4) % 4) * 128, 128)],
                    sems[(g + 4) % 4])

        pltpu.sync_copy(ev1_v, acc.at[ii1_v], add=True)

    plsc.subcore_barrier()
    pltpu.sync_copy(acc.at[pl.ds(sid * ASL, ASL)], zbuf)
    pltpu.sync_copy(zbuf, epart.at[pl.ds(cid * NAP + sid * ASL, ASL)])


def _combine_body(a_ref, o_ref):
    o_ref[...] = a_ref[0] + a_ref[1]


def _combine(parts):
    out = pl.pallas_call(
        _combine_body,
        out_shape=jax.ShapeDtypeStruct((NAP // 128, 128), jnp.float32),
    )(parts.reshape(2, NAP // 128, 128))
    return out.reshape(NAP)


@jax.jit
def kernel(atomic_numbers, distances, idx_i, idx_j, c6ab, rcov, r2r4,
           d3_s6, d3_s8, d3_a1, d3_a2):
    f32 = jnp.float32
    i32 = jnp.int32
    pad_p = NPP - N_PAIRS
    ii = jnp.pad(idx_i.astype(i32), (0, pad_p)).reshape(PROWS, 128)
    jj = jnp.pad(idx_j.astype(i32), (0, pad_p)).reshape(PROWS, 128)
    d = jnp.pad(distances.astype(f32), (0, pad_p),
                constant_values=11.0).reshape(PROWS, 128)
    atoms = jnp.pad(atomic_numbers.astype(i32), (0, NAP - N_ATOMS))
    rcov96 = jnp.pad(rcov.astype(f32), (0, 96 - N_ELEM))
    u96 = jnp.pad((3.0 ** 0.25) * jnp.sqrt(r2r4.astype(f32)),
                  (0, 96 - N_ELEM))
    c6f = c6ab.astype(f32)
    cn0 = c6f[..., 0].reshape(N_ELEM * N_ELEM, 25)
    u32 = jnp.uint32
    w1 = lax.bitcast_convert_type(c6f[..., 1].astype(jnp.bfloat16), jnp.uint16
                                  ).reshape(N_ELEM * N_ELEM, 25).astype(u32)
    w2 = lax.bitcast_convert_type(c6f[..., 2].astype(jnp.bfloat16), jnp.uint16
                                  ).reshape(N_ELEM * N_ELEM, 25).astype(u32)
    packed = lax.bitcast_convert_type(w1 | (w2 << 16), i32)
    table = jnp.concatenate(
        [lax.bitcast_convert_type(cn0, i32), packed,
         jnp.zeros((N_ELEM * N_ELEM, TROW - 50), i32)], axis=1)
    table = jnp.pad(table, ((0, NTROWS - N_ELEM * N_ELEM), (0, 0)))
    params = jnp.pad(jnp.stack([d3_s6, d3_s8, d3_a1, d3_a2]).astype(f32),
                     (0, 12))

    mesh = plsc.VectorSubcoreMesh(core_axis_name="c", subcore_axis_name="s",
                                  num_cores=NC, num_subcores=NS)
    cparams = pltpu.CompilerParams(needs_layout_passes=False,
                                   use_tc_tiling_on_sc=False)

    p1 = pl.kernel(
        _phase1_body,
        out_type=[
            jax.ShapeDtypeStruct((NC * NAP,), f32),
            jax.ShapeDtypeStruct((PROWS, 128), i32),
            jax.ShapeDtypeStruct((PROWS, 128), f32),
        ],
        mesh=mesh,
        compiler_params=cparams,
        scratch_types=[
            pltpu.VMEM((NAP,), i32),
            pltpu.VMEM((96,), f32),
            pltpu.VMEM((96,), f32),
            pltpu.VMEM((R1, 128), i32),
            pltpu.VMEM((R1, 128), i32),
            pltpu.VMEM((R1, 128), f32),
            pltpu.VMEM((C1,), i32),
            pltpu.VMEM((C1,), f32),
            pltpu.VMEM((R1, 128), i32),
            pltpu.VMEM((R1, 128), f32),
            pltpu.VMEM((ASL,), f32),
            pltpu.VMEM_SHARED((NAP,), f32),
        ],
    )
    ncpart, cls, srr = p1(atoms, ii, jj, d, rcov96, u96)
    nc = _combine(ncpart)

    p2 = pl.kernel(
        _phase2_body,
        out_type=jax.ShapeDtypeStruct((NC * NAP,), f32),
        mesh=mesh,
        compiler_params=cparams,
        scratch_types=[
            pltpu.VMEM((NAP,), f32),
            pltpu.VMEM((16,), f32),
            pltpu.VMEM((R2, 128), i32),
            pltpu.VMEM((R2, 128), i32),
            pltpu.VMEM((R2, 128), f32),
            pltpu.VMEM((R2, 128), f32),
            pltpu.VMEM((C2,), i32),
            pltpu.VMEM((C2,), f32),
            pltpu.VMEM((R2, 128), i32),
            pltpu.VMEM((512, TROW), i32),
            pltpu.VMEM((ASL,), f32),
            pltpu.SemaphoreType.DMA,
            pltpu.SemaphoreType.DMA,
            pltpu.SemaphoreType.DMA,
            pltpu.SemaphoreType.DMA,
            pltpu.VMEM_SHARED((NTROWS, TROW), i32),
            pltpu.VMEM_SHARED((NAP,), f32),
        ],
    )
    epart = p2(table, nc, ii, jj, d, srr, cls, params)
    return _combine(epart)[:N_ATOMS]


# speedup vs baseline: 571.0871x; 1.0607x over previous
import jax
import jax.numpy as jnp
from jax import lax
from jax.experimental import pallas as pl
from jax.experimental.pallas import tpu as pltpu
from jax.experimental.pallas import tpu_sc as plsc

N_ATOMS = 50000
N_PAIRS = 1600000
N_ELEM = 95
CUTOFF = 10.0
WIDTH = 2.0
CUTON = CUTOFF - WIDTH
K1 = 16.0
K3 = -4.0

NC = 2
NS = 16
NT = NC * NS
NAP = 50176
ASL = NAP // NS
PT = 50176
NPP = NT * PT
PROWS = NPP // 128

R1 = 56
C1 = R1 * 128
R2 = 8
C2 = R2 * 128
TROW = 56
NTROWS = 9216


def _smoother(d):
    x = (CUTOFF - d) * (1.0 / WIDTH)
    poly = ((6.0 * x - 15.0) * x + 10.0) * x * x * x
    return jnp.where(d < CUTON, 1.0, jnp.where(d >= CUTOFF, 0.0, poly))


def _phase1_body(atoms_h, ii_h, jj_h, d_h, rcov_h, u_h,
                 ncpart, cls_o, srr_o,
                 atoms_v, rcov_v, u_v, ii_v, jj_v, d_v,
                 ii1_v, vals1_v, cls_v, srr_v, zbuf, acc):
    cid = lax.axis_index("c")
    sid = lax.axis_index("s")
    wid = cid * NS + sid

    pltpu.sync_copy(atoms_h, atoms_v)
    pltpu.sync_copy(rcov_h, rcov_v)
    pltpu.sync_copy(u_h, u_v)

    @pl.loop(0, ASL // 16)
    def _z(k):
        zbuf[pl.ds(k * 16, 16)] = jnp.zeros((16,), jnp.float32)

    pltpu.sync_copy(zbuf, acc.at[pl.ds(sid * ASL, ASL)])
    plsc.subcore_barrier()

    row_base = wid * (PT // 128)

    @pl.loop(0, PT // C1)
    def _chunk(c):
        roff = row_base + c * R1
        pltpu.sync_copy(ii_h.at[pl.ds(roff, R1)], ii_v)
        pltpu.sync_copy(jj_h.at[pl.ds(roff, R1)], jj_v)
        pltpu.sync_copy(d_h.at[pl.ds(roff, R1)], d_v)

        @pl.loop(0, R1)
        def _row(g):
            @pl.loop(0, 8)
            def _grp(k):
                s = k * 16
                ii = ii_v[g, pl.ds(s, 16)]
                jj = jj_v[g, pl.ds(s, 16)]
                d = d_v[g, pl.ds(s, 16)]
                zi = plsc.load_gather(atoms_v, [ii])
                zj = plsc.load_gather(atoms_v, [jj])
                rco = (plsc.load_gather(rcov_v, [zi]) +
                       plsc.load_gather(rcov_v, [zj]))
                damp = 1.0 / (1.0 + jnp.exp(-K1 * (rco / d - 1.0)))
                ii1_v[pl.ds(g * 128 + s, 16)] = ii
                vals1_v[pl.ds(g * 128 + s, 16)] = damp * _smoother(d)
                cls_v[g, pl.ds(s, 16)] = zi * N_ELEM + zj
                srr_v[g, pl.ds(s, 16)] = (plsc.load_gather(u_v, [zi]) *
                                          plsc.load_gather(u_v, [zj]))

        pltpu.sync_copy(vals1_v, acc.at[ii1_v], add=True)
        pltpu.sync_copy(cls_v, cls_o.at[pl.ds(roff, R1)])
        pltpu.sync_copy(srr_v, srr_o.at[pl.ds(roff, R1)])

    plsc.subcore_barrier()
    pltpu.sync_copy(acc.at[pl.ds(sid * ASL, ASL)], zbuf)
    pltpu.sync_copy(zbuf, ncpart.at[pl.ds(cid * NAP + sid * ASL, ASL)])


def _phase2_body(table_h, nc_h, ii_h, jj_h, d_h, srr_h, cls_h, params_h,
                 epart,
                 nc_v, params_v, ii_v, jj_v, d_v, srr_v, ii1_v, ev1_v,
                 cls_v, rows_v, zbuf, sem0, sem1, sem2, sem3, spm_table, acc):
    cid = lax.axis_index("c")
    sid = lax.axis_index("s")
    wid = cid * NS + sid

    pltpu.sync_copy(nc_h, nc_v)
    pltpu.sync_copy(params_h, params_v)

    @pl.loop(0, ASL // 16)
    def _z(k):
        zbuf[pl.ds(k * 16, 16)] = jnp.zeros((16,), jnp.float32)

    pltpu.sync_copy(zbuf, acc.at[pl.ds(sid * ASL, ASL)])

    tb = sid * (NTROWS // NS)

    @pl.loop(0, NTROWS // NS // 64)
    def _stage(j):
        pltpu.sync_copy(table_h.at[pl.ds(tb + j * 64, 64)],
                        rows_v.at[pl.ds(0, 64)])
        pltpu.sync_copy(rows_v.at[pl.ds(0, 64)],
                        spm_table.at[pl.ds(tb + j * 64, 64)])

    plsc.subcore_barrier()

    pv = params_v[pl.ds(0, 16)]
    s6 = pv[0]
    s8 = pv[1]
    a1 = pv[2]
    a2 = pv[3]
    row_base = wid * (PT // 128)
    iota = lax.iota(jnp.int32, 16)

    @pl.loop(0, PT // C2)
    def _chunk(c):
        roff = row_base + c * R2
        pltpu.sync_copy(cls_h.at[pl.ds(roff, R2)], cls_v)
        pltpu.sync_copy(ii_h.at[pl.ds(roff, R2)], ii_v)
        pltpu.sync_copy(jj_h.at[pl.ds(roff, R2)], jj_v)
        pltpu.sync_copy(d_h.at[pl.ds(roff, R2)], d_v)
        pltpu.sync_copy(srr_h.at[pl.ds(roff, R2)], srr_v)
        sems = [sem0, sem1, sem2, sem3]
        dsc = [None] * R2
        for g in range(4):
            dsc[g] = pltpu.async_copy(spm_table.at[cls_v.at[g]],
                                      rows_v.at[pl.ds((g % 4) * 128, 128)],
                                      sems[g % 4])
        for g in range(R2):
            dsc[g].wait()

            @pl.loop(0, 8)
            def _grp(k):
                s = k * 16
                ii = ii_v[g, pl.ds(s, 16)]
                jj = jj_v[g, pl.ds(s, 16)]
                d = d_v[g, pl.ds(s, 16)]
                srr = srr_v[g, pl.ds(s, 16)]
                ii1_v[pl.ds(g * 128 + s, 16)] = ii
                nci = plsc.load_gather(nc_v, [ii])
                ncj = plsc.load_gather(nc_v, [jj])
                pr = (g % 4) * 128 + s + iota
                rsum = jnp.zeros((16,), jnp.float32)
                csum = jnp.zeros((16,), jnp.float32)
                for t in range(25):
                    c0 = plsc.bitcast(
                        plsc.load_gather(rows_v, [pr, jnp.full((16,), t, jnp.int32)]),
                        jnp.float32)
                    w12 = plsc.load_gather(rows_v, [pr, jnp.full((16,), 25 + t, jnp.int32)])
                    c1 = plsc.bitcast(lax.shift_left(w12, 16), jnp.float32)
                    c2 = plsc.bitcast(w12 & jnp.int32(-65536), jnp.float32)
                    dr1 = c1 - nci
                    dr2 = c2 - ncj
                    w = jnp.exp(K3 * (dr1 * dr1 + dr2 * dr2))
                    m = c0 > 0.0
                    rsum = rsum + jnp.where(m, w, 0.0)
                    csum = csum + jnp.where(m, w * c0, 0.0)
                c6 = jnp.where(rsum > 0.0, csum / rsum, -1e38)
                c8 = c6 * srr * srr
                tmp = a1 * srr + a2
                tmp2 = tmp * tmp
                tmp6 = tmp2 * tmp2 * tmp2
                tmp8 = tmp6 * tmp2
                d2 = d * d
                d6 = d2 * d2 * d2
                d8 = d6 * d2
                sw = _smoother(d)
                ev1_v[pl.ds(g * 128 + s, 16)] = -0.5 * sw * (s6 * c6 / (d6 + tmp6) +
                                                             s8 * c8 / (d8 + tmp8))

            if g + 4 < R2:
                dsc[g + 4] = pltpu.async_copy(
                    spm_table.at[cls_v.at[g + 4]],
                    rows_v.at[pl.ds(((g + 4) % 4) * 128, 128)],
                    sems[(g + 4) % 4])

        pltpu.sync_copy(ev1_v, acc.at[ii1_v], add=True)

    plsc.subcore_barrier()
    pltpu.sync_copy(acc.at[pl.ds(sid * ASL, ASL)], zbuf)
    pltpu.sync_copy(zbuf, epart.at[pl.ds(cid * NAP + sid * ASL, ASL)])


def _combine_body(a_ref, o_ref):
    o_ref[...] = a_ref[0] + a_ref[1]


def _combine(parts):
    out = pl.pallas_call(
        _combine_body,
        out_shape=jax.ShapeDtypeStruct((NAP // 128, 128), jnp.float32),
    )(parts.reshape(2, NAP // 128, 128))
    return out.reshape(NAP)


@jax.jit
def kernel(atomic_numbers, distances, idx_i, idx_j, c6ab, rcov, r2r4,
           d3_s6, d3_s8, d3_a1, d3_a2):
    f32 = jnp.float32
    i32 = jnp.int32
    pad_p = NPP - N_PAIRS
    ii = jnp.pad(idx_i.astype(i32), (0, pad_p)).reshape(PROWS, 128)
    jj = jnp.pad(idx_j.astype(i32), (0, pad_p)).reshape(PROWS, 128)
    d = jnp.pad(distances.astype(f32), (0, pad_p),
                constant_values=11.0).reshape(PROWS, 128)
    atoms = jnp.pad(atomic_numbers.astype(i32), (0, NAP - N_ATOMS))
    rcov96 = jnp.pad(rcov.astype(f32), (0, 96 - N_ELEM))
    u96 = jnp.pad((3.0 ** 0.25) * jnp.sqrt(r2r4.astype(f32)),
                  (0, 96 - N_ELEM))
    c6f = c6ab.astype(f32)
    cn0 = c6f[..., 0].reshape(N_ELEM * N_ELEM, 25)
    u32 = jnp.uint32
    w1 = lax.bitcast_convert_type(c6f[..., 1].astype(jnp.bfloat16), jnp.uint16
                                  ).reshape(N_ELEM * N_ELEM, 25).astype(u32)
    w2 = lax.bitcast_convert_type(c6f[..., 2].astype(jnp.bfloat16), jnp.uint16
                                  ).reshape(N_ELEM * N_ELEM, 25).astype(u32)
    packed = lax.bitcast_convert_type(w1 | (w2 << 16), i32)
    table = jnp.concatenate(
        [lax.bitcast_convert_type(cn0, i32), packed,
         jnp.zeros((N_ELEM * N_ELEM, TROW - 50), i32)], axis=1)
    table = jnp.pad(table, ((0, NTROWS - N_ELEM * N_ELEM), (0, 0)))
    params = jnp.pad(jnp.stack([d3_s6, d3_s8, d3_a1, d3_a2]).astype(f32),
                     (0, 12))

    mesh = plsc.VectorSubcoreMesh(core_axis_name="c", subcore_axis_name="s",
                                  num_cores=NC, num_subcores=NS)
    cparams = pltpu.CompilerParams(needs_layout_passes=False,
                                   use_tc_tiling_on_sc=False)

    p1 = pl.kernel(
        _phase1_body,
        out_type=[
            jax.ShapeDtypeStruct((NC * NAP,), f32),
            jax.ShapeDtypeStruct((PROWS, 128), i32),
            jax.ShapeDtypeStruct((PROWS, 128), f32),
        ],
        mesh=mesh,
        compiler_params=cparams,
        scratch_types=[
            pltpu.VMEM((NAP,), i32),
            pltpu.VMEM((96,), f32),
            pltpu.VMEM((96,), f32),
            pltpu.VMEM((R1, 128), i32),
            pltpu.VMEM((R1, 128), i32),
            pltpu.VMEM((R1, 128), f32),
            pltpu.VMEM((C1,), i32),
            pltpu.VMEM((C1,), f32),
            pltpu.VMEM((R1, 128), i32),
            pltpu.VMEM((R1, 128), f32),
            pltpu.VMEM((ASL,), f32),
            pltpu.VMEM_SHARED((NAP,), f32),
        ],
    )
    ncpart, cls, srr = p1(atoms, ii, jj, d, rcov96, u96)
    nc = _combine(ncpart)

    p2 = pl.kernel(
        _phase2_body,
        out_type=jax.ShapeDtypeStruct((NC * NAP,), f32),
        mesh=mesh,
        compiler_params=cparams,
        scratch_types=[
            pltpu.VMEM((NAP,), f32),
            pltpu.VMEM((16,), f32),
            pltpu.VMEM((R2, 128), i32),
            pltpu.VMEM((R2, 128), i32),
            pltpu.VMEM((R2, 128), f32),
            pltpu.VMEM((R2, 128), f32),
            pltpu.VMEM((C2,), i32),
            pltpu.VMEM((C2,), f32),
            pltpu.VMEM((R2, 128), i32),
            pltpu.VMEM((512, TROW), i32),
            pltpu.VMEM((ASL,), f32),
            pltpu.SemaphoreType.DMA,
            pltpu.SemaphoreType.DMA,
            pltpu.SemaphoreType.DMA,
            pltpu.SemaphoreType.DMA,
            pltpu.VMEM_SHARED((NTROWS, TROW), i32),
            pltpu.VMEM_SHARED((NAP,), f32),
        ],
    )
    epart = p2(table, nc, ii, jj, d, srr, cls, params)
    return _combine(epart)[:N_ATOMS]
